# baseline (speedup 1.0000x reference)
import functools

import jax
import jax.numpy as jnp
from jax import lax
from jax.experimental import pallas as pl
from jax.experimental.pallas import tpu as pltpu
from jax.experimental.pallas import tpu_sc as plsc

D_MODEL = 64
SCALE = float(D_MODEL) ** 0.5

_INFO = plsc.get_sparse_core_info()
_NC = _INFO.num_cores
_NS = _INFO.num_subcores
_NW = _NC * _NS
_LANES = _INFO.num_lanes

_IW = 128
_IDX_ROWS = 8
_SUB = 2
_CHUNK = _SUB * _IW


@functools.partial(jax.jit, static_argnames=("n_rows",))
def _embed(x2d, t4, n_rows):
    n_idx_rows = x2d.shape[0]
    rows_per_w = n_rows // _NW
    idx_rows_per_w = n_idx_rows // _NW
    chunks = idx_rows_per_w // _IDX_ROWS

    mesh = plsc.VectorSubcoreMesh(core_axis_name="c", subcore_axis_name="s")

    @functools.partial(
        pl.kernel,
        mesh=mesh,
        out_type=jax.ShapeDtypeStruct((n_rows // 2, 2 * D_MODEL), jnp.float32),
        scratch_types=[
            pltpu.VMEM((_IDX_ROWS, _IW), jnp.int32),
            pltpu.VMEM((_CHUNK, 2 * D_MODEL), jnp.float32),
            pltpu.VMEM((_CHUNK // 2, 2 * D_MODEL), jnp.float32),
            pltpu.SemaphoreType.DMA,
        ],
    )
    def k(x_hbm, t4_hbm, out_hbm, idx_v, rows_v, pack_v, gsem):
        wid = lax.axis_index("s") * _NC + lax.axis_index("c")
        idx_row0 = wid * idx_rows_per_w
        pair_base = wid * (rows_per_w // 2)

        def chunk_body(t, _):
            pltpu.sync_copy(
                x_hbm.at[pl.ds(idx_row0 + t * _IDX_ROWS, _IDX_ROWS)], idx_v
            )
            for s in range(_IDX_ROWS // _SUB):
                descs = []
                for j in range(_SUB):
                    descs.append(
                        pltpu.async_copy(
                            t4_hbm.at[idx_v.at[s * _SUB + j]],
                            rows_v.at[pl.ds(j * _IW, _IW)],
                            gsem,
                        )
                    )
                for d in descs:
                    d.wait()

                def pack_body(p, _):
                    for rr in range(2):
                        for c in range(D_MODEL // _LANES):
                            src = pl.ds(c * _LANES, _LANES)
                            dst = pl.ds(rr * D_MODEL + c * _LANES, _LANES)
                            pack_v[p, dst] = rows_v[2 * p + rr, src]
                    return ()

                lax.fori_loop(0, _CHUNK // 2, pack_body, ())

                pltpu.sync_copy(
                    pack_v,
                    out_hbm.at[
                        pl.ds(pair_base
                              + (t * (_IDX_ROWS // _SUB) + s) * (_CHUNK // 2),
                              _CHUNK // 2)
                    ],
                )
            return ()

        lax.fori_loop(0, chunks, chunk_body, ())

    return k(x2d, t4)


def kernel(x, table):
    b0, b1 = x.shape
    n_rows = b0 * b1
    x2d = x.reshape(n_rows // _IW, _IW).astype(jnp.int32)
    ts = table * SCALE
    t4 = jnp.concatenate([ts, ts], axis=1)
    out = _embed(x2d, t4, n_rows)
    return out.reshape(n_rows, D_MODEL).reshape(b0, b1, D_MODEL)


# speedup vs baseline: 1.5570x; 1.5570x over previous
import functools

import jax
import jax.numpy as jnp
from jax import lax
from jax.experimental import pallas as pl
from jax.experimental.pallas import tpu as pltpu
from jax.experimental.pallas import tpu_sc as plsc

D_MODEL = 64
SCALE = float(D_MODEL) ** 0.5

_INFO = plsc.get_sparse_core_info()
_NC = _INFO.num_cores
_NS = _INFO.num_subcores
_NW = _NC * _NS
_LANES = _INFO.num_lanes

_IW = 128
_GRP = 2
_CHUNK = _GRP * _IW
_STAGE = 1024


@functools.partial(jax.jit, static_argnames=("n_rows",))
def _embed(x1, table, n_rows):
    rows_per_w = n_rows // _NW
    chunks = rows_per_w // _STAGE

    mesh = plsc.VectorSubcoreMesh(core_axis_name="c", subcore_axis_name="s")

    @functools.partial(
        pl.kernel,
        mesh=mesh,
        out_type=jax.ShapeDtypeStruct((n_rows // 2, 2 * D_MODEL), jnp.float32),
        scratch_types=[
            pltpu.VMEM((_STAGE,), jnp.int32),
            pltpu.VMEM((_CHUNK, D_MODEL), jnp.float32),
            pltpu.VMEM((_CHUNK // 2, 2 * D_MODEL), jnp.float32),
            pltpu.SemaphoreType.DMA,
        ],
        compiler_params=pltpu.CompilerParams(use_tc_tiling_on_sc=False),
    )
    def k(x_hbm, table_hbm, out_hbm, idx_v, rows_v, pack_v, gsem):
        wid = lax.axis_index("s") * _NC + lax.axis_index("c")
        base = wid * rows_per_w
        pair_base = wid * (rows_per_w // 2)

        def chunk_body(t, _):
            pltpu.sync_copy(x_hbm.at[pl.ds(base + t * _STAGE, _STAGE)], idx_v)
            for s in range(_STAGE // _CHUNK):
                descs = []
                for j in range(_GRP):
                    descs.append(
                        pltpu.async_copy(
                            table_hbm.at[
                                idx_v.at[pl.ds((s * _GRP + j) * _IW, _IW)]
                            ],
                            rows_v.at[pl.ds(j * _IW, _IW)],
                            gsem,
                        )
                    )
                for d in descs:
                    d.wait()

                def pack_body(p, _):
                    for rr in range(2):
                        for c in range(D_MODEL // _LANES):
                            src = pl.ds(c * _LANES, _LANES)
                            dst = pl.ds(rr * D_MODEL + c * _LANES, _LANES)
                            pack_v[p, dst] = rows_v[2 * p + rr, src] * SCALE
                    return ()

                lax.fori_loop(0, _CHUNK // 2, pack_body, ())

                pltpu.sync_copy(
                    pack_v,
                    out_hbm.at[
                        pl.ds(pair_base + (t * (_STAGE // _CHUNK) + s)
                              * (_CHUNK // 2),
                              _CHUNK // 2)
                    ],
                )
            return ()

        lax.fori_loop(0, chunks, chunk_body, ())

    return k(x1, table)


def kernel(x, table):
    b0, b1 = x.shape
    n_rows = b0 * b1
    x1 = x.reshape(n_rows).astype(jnp.int32)
    out = _embed(x1, table, n_rows)
    return out.reshape(b0, b1, D_MODEL)
